# baseline (speedup 1.0000x reference)
import functools

import jax
import jax.numpy as jnp
from jax import lax
from jax.experimental import pallas as pl
from jax.experimental.pallas import tpu as pltpu
from jax.experimental.pallas import tpu_sc as plsc

N_EXP = 8
BLK_T = 2048
NC = 2
NS = 16
NW = NC * NS
LANES = 16
PHASES = 4


def _mm_kernel(x_ref, w_ref, st_ref):
    st_ref[...] = jax.lax.dot_general(
        w_ref[...], x_ref[...], (((1,), (1,)), ((), ())),
        preferred_element_type=jnp.float32,
    )


def _scores_t(x, W, phase, n_phase):
    n_tokens, dim = x.shape
    blocks = n_phase // BLK_T
    return pl.pallas_call(
        _mm_kernel,
        grid=(blocks,),
        in_specs=[
            pl.BlockSpec(
                (BLK_T, dim), lambda i, p=phase, b=blocks: (p * b + i, 0)
            ),
            pl.BlockSpec((N_EXP, dim), lambda i: (0, 0)),
        ],
        out_specs=pl.BlockSpec((N_EXP, BLK_T), lambda i: (0, i)),
        out_shape=jax.ShapeDtypeStruct((N_EXP, n_phase), jnp.float32),
    )(x, W)


def _make_route(n_phase):
    chunk = n_phase // NW

    @functools.partial(
        pl.kernel,
        mesh=plsc.VectorSubcoreMesh(core_axis_name="c", subcore_axis_name="s"),
        out_type=[
            jax.ShapeDtypeStruct((n_phase,), jnp.float32),
            jax.ShapeDtypeStruct((n_phase,), jnp.float32),
            jax.ShapeDtypeStruct((n_phase,), jnp.int32),
            jax.ShapeDtypeStruct((n_phase,), jnp.int32),
        ],
        scratch_types=[
            pltpu.VMEM((N_EXP, chunk), jnp.float32),
            pltpu.VMEM((2, chunk), jnp.float32),
            pltpu.VMEM((2, chunk), jnp.int32),
        ],
    )
    def route(st_hbm, w1_hbm, w2_hbm, i1_hbm, i2_hbm, s_v, w_v, i_v):
        wid = lax.axis_index("s") * NC + lax.axis_index("c")
        base = wid * chunk
        for e in range(N_EXP):
            pltpu.sync_copy(
                st_hbm.at[pl.ds(e * n_phase + base, chunk)], s_v.at[e]
            )

        def body(t, _):
            off = t * LANES
            vs = [s_v[e, pl.ds(off, LANES)] for e in range(N_EXP)]
            m1 = vs[0]
            i1 = jnp.zeros((LANES,), jnp.int32)
            m2 = jnp.full((LANES,), -jnp.inf, jnp.float32)
            i2 = jnp.zeros((LANES,), jnp.int32)
            for e in range(1, N_EXP):
                v = vs[e]
                ev = jnp.full((LANES,), e, jnp.int32)
                gt1 = v > m1
                gt2 = v > m2
                m2n = jnp.where(gt1, m1, jnp.where(gt2, v, m2))
                i2n = jnp.where(gt1, i1, jnp.where(gt2, ev, i2))
                m1 = jnp.where(gt1, v, m1)
                i1 = jnp.where(gt1, ev, i1)
                m2, i2 = m2n, i2n
            denom = jnp.zeros((LANES,), jnp.float32)
            for e in range(N_EXP):
                denom = denom + jnp.exp(vs[e] - m1)
            w1 = 1.0 / denom
            w2 = jnp.exp(m2 - m1) * w1
            sl = pl.ds(off, LANES)
            w_v[0, sl] = w1
            w_v[1, sl] = w2
            i_v[0, sl] = i1
            i_v[1, sl] = i2
            return 0

        lax.fori_loop(0, chunk // LANES, body, 0)
        rows = pl.ds(base, chunk)
        pltpu.sync_copy(w_v.at[0], w1_hbm.at[rows])
        pltpu.sync_copy(w_v.at[1], w2_hbm.at[rows])
        pltpu.sync_copy(i_v.at[0], i1_hbm.at[rows])
        pltpu.sync_copy(i_v.at[1], i2_hbm.at[rows])

    return route


@jax.jit
def kernel(x, W):
    n_tokens, _ = x.shape
    n_phase = n_tokens // PHASES
    route = _make_route(n_phase)
    parts = []
    for p in range(PHASES):
        st = _scores_t(x, W, p, n_phase)
        parts.append(route(st.reshape(-1)))
    w1 = jnp.concatenate([q[0] for q in parts])
    w2 = jnp.concatenate([q[1] for q in parts])
    i1 = jnp.concatenate([q[2] for q in parts])
    i2 = jnp.concatenate([q[3] for q in parts])
    return jnp.stack([w1, w2], axis=1), jnp.stack([i1, i2], axis=1)


# speedup vs baseline: 1.1605x; 1.1605x over previous
import functools

import jax
import jax.numpy as jnp
from jax import lax
from jax.experimental import pallas as pl
from jax.experimental.pallas import tpu as pltpu
from jax.experimental.pallas import tpu_sc as plsc

N_EXP = 8
BLK_T = 2048
NC = 2
NS = 16
NW = NC * NS
LANES = 16


def _mm_kernel(x_ref, w_ref, st_ref):
    st_ref[...] = jax.lax.dot_general(
        w_ref[...], x_ref[...], (((1,), (1,)), ((), ())),
        preferred_element_type=jnp.float32,
    )


def _scores_t(x, W):
    n_tokens, dim = x.shape
    return pl.pallas_call(
        _mm_kernel,
        grid=(n_tokens // BLK_T,),
        in_specs=[
            pl.BlockSpec((BLK_T, dim), lambda i: (i, 0)),
            pl.BlockSpec((N_EXP, dim), lambda i: (0, 0)),
        ],
        out_specs=pl.BlockSpec((N_EXP, BLK_T), lambda i: (0, i)),
        out_shape=jax.ShapeDtypeStruct((N_EXP, n_tokens), jnp.float32),
    )(x, W)


def _make_route(n_tokens):
    chunk = n_tokens // NW

    @functools.partial(
        pl.kernel,
        mesh=plsc.VectorSubcoreMesh(core_axis_name="c", subcore_axis_name="s"),
        out_type=[
            jax.ShapeDtypeStruct((n_tokens,), jnp.float32),
            jax.ShapeDtypeStruct((n_tokens,), jnp.float32),
            jax.ShapeDtypeStruct((n_tokens,), jnp.int32),
            jax.ShapeDtypeStruct((n_tokens,), jnp.int32),
        ],
        scratch_types=[
            pltpu.VMEM((N_EXP, chunk), jnp.float32),
            pltpu.VMEM((2, chunk), jnp.float32),
            pltpu.VMEM((2, chunk), jnp.int32),
            pltpu.SemaphoreType.DMA,
            pltpu.SemaphoreType.DMA,
        ],
    )
    def route(st_hbm, w1_hbm, w2_hbm, i1_hbm, i2_hbm, s_v, w_v, i_v,
              in_sem, out_sem):
        wid = lax.axis_index("s") * NC + lax.axis_index("c")
        base = wid * chunk
        copies = [
            pltpu.async_copy(
                st_hbm.at[pl.ds(e * n_tokens + base, chunk)], s_v.at[e], in_sem
            )
            for e in range(N_EXP)
        ]
        for c in copies:
            c.wait()

        def body(t, _):
            off = t * LANES
            vs = [s_v[e, pl.ds(off, LANES)] for e in range(N_EXP)]
            m1 = vs[0]
            i1 = jnp.zeros((LANES,), jnp.int32)
            m2 = jnp.full((LANES,), -jnp.inf, jnp.float32)
            i2 = jnp.zeros((LANES,), jnp.int32)
            for e in range(1, N_EXP):
                v = vs[e]
                ev = jnp.full((LANES,), e, jnp.int32)
                gt1 = v > m1
                gt2 = v > m2
                m2n = jnp.where(gt1, m1, jnp.where(gt2, v, m2))
                i2n = jnp.where(gt1, i1, jnp.where(gt2, ev, i2))
                m1 = jnp.where(gt1, v, m1)
                i1 = jnp.where(gt1, ev, i1)
                m2, i2 = m2n, i2n
            denom = jnp.zeros((LANES,), jnp.float32)
            for e in range(N_EXP):
                denom = denom + jnp.exp(vs[e] - m1)
            w1 = 1.0 / denom
            w2 = jnp.exp(m2 - m1) * w1
            sl = pl.ds(off, LANES)
            w_v[0, sl] = w1
            w_v[1, sl] = w2
            i_v[0, sl] = i1
            i_v[1, sl] = i2
            return 0

        lax.fori_loop(0, chunk // LANES, body, 0)
        rows = pl.ds(base, chunk)
        outs = [
            pltpu.async_copy(w_v.at[0], w1_hbm.at[rows], out_sem),
            pltpu.async_copy(w_v.at[1], w2_hbm.at[rows], out_sem),
            pltpu.async_copy(i_v.at[0], i1_hbm.at[rows], out_sem),
            pltpu.async_copy(i_v.at[1], i2_hbm.at[rows], out_sem),
        ]
        for c in outs:
            c.wait()

    return route


@jax.jit
def kernel(x, W):
    n_tokens, _ = x.shape
    st = _scores_t(x, W)
    w1, w2, i1, i2 = _make_route(n_tokens)(st.reshape(-1))
    return jnp.stack([w1, w2], axis=1), jnp.stack([i1, i2], axis=1)
